# baseline (speedup 1.0000x reference)
import jax
import jax.numpy as jnp
from jax import lax
from jax.experimental import pallas as pl
from jax.experimental.pallas import tpu as pltpu
from jax.experimental.pallas import tpu_sc as plsc

N_NODES = 10000
NUM_GRAPHS = 4
N_EDGES = 320000
F = 16

NC, NS = 2, 16
NW = NC * NS
BLK = 128
K = -(-N_EDGES // (NW * BLK * 8)) * 8
ROWS = K * NW
E_PAD = ROWS * BLK
NPAD = 10240
ZPT = NPAD // NS

_MESH = plsc.VectorSubcoreMesh(
    core_axis_name="c", subcore_axis_name="s", num_cores=NC, num_subcores=NS
)


CHB = 512
NBLK = K * BLK // CHB
NBUF = 2
NRING = 2 * NBUF
KS = NBLK // NRING


def _edge_body(table, src3d, dst3d, zeros, out, src_v, dst_v, bufs, acc,
               tab_sh, gsem, ssem):
    c = lax.axis_index("c")
    s = lax.axis_index("s")
    wid = c * NS + s
    z0 = s * ZPT
    pltpu.sync_copy(zeros.at[pl.ds(z0, ZPT)], acc.at[pl.ds(z0, ZPT)])
    pltpu.sync_copy(table.at[pl.ds(z0, ZPT)], tab_sh.at[pl.ds(z0, ZPT)])
    pltpu.sync_copy(src3d.at[wid], src_v)
    pltpu.sync_copy(dst3d.at[wid], dst_v)
    plsc.subcore_barrier()

    def gather(j, b):
        pltpu.async_copy(tab_sh.at[src_v.at[j]], bufs.at[b], gsem.at[b])

    def gather_wait(j, b):
        pltpu.make_async_copy(tab_sh.at[src_v.at[j]], bufs.at[b],
                              gsem.at[b]).wait()

    def scatter(j, b):
        pltpu.async_copy(bufs.at[b], acc.at[dst_v.at[j]], ssem.at[b],
                         add=True)

    def scatter_wait(j, b):
        pltpu.make_async_copy(bufs.at[b], acc.at[dst_v.at[j]],
                              ssem.at[b]).wait()

    for b in range(NBUF):
        gather(b, b)

    def step(sb, carry):
        for b in range(NRING):
            j = sb * NRING + b
            gather_wait(j, b)
            scatter(j, b)
            b4 = (b + NBUF) % NRING

            @pl.when(j >= NBUF)
            def _drain(j=j, b4=b4):
                scatter_wait(j - NBUF, b4)

            @pl.when(j + NBUF < NBLK)
            def _prefetch(j=j, b4=b4):
                gather(j + NBUF, b4)
        return carry

    lax.fori_loop(0, KS, step, 0)
    for b in range(NBUF):
        j = (KS - 1) * NRING + NBUF + b
        scatter_wait(j, (j % NRING))
    plsc.subcore_barrier()
    pltpu.sync_copy(acc.at[pl.ds(z0, ZPT)], out.at[pl.ds(c * NPAD + z0, ZPT)])


NSEM4 = 4
KS4 = NBLK // NSEM4


def _deg_body(ones, dst3d, zeros, out, dst_v, rows_v, acc, sems):
    c = lax.axis_index("c")
    s = lax.axis_index("s")
    wid = c * NS + s
    z0 = s * ZPT
    pltpu.sync_copy(zeros.at[pl.ds(z0, ZPT)], acc.at[pl.ds(z0, ZPT)])
    pltpu.sync_copy(dst3d.at[wid], dst_v)
    pltpu.sync_copy(ones, rows_v)
    plsc.subcore_barrier()

    def didx(j):
        return acc.at[dst_v.at[j]]

    def step(sb, carry):
        for b in range(NSEM4):
            j = sb * NSEM4 + b

            @pl.when(sb > 0)
            def _drain(j=j, b=b):
                pltpu.make_async_copy(rows_v, didx(j - NSEM4),
                                      sems.at[b]).wait()

            pltpu.async_copy(rows_v, didx(j), sems.at[b], add=True)
        return carry

    lax.fori_loop(0, KS4, step, 0)
    for b in range(NSEM4):
        pltpu.make_async_copy(rows_v, didx((KS4 - 1) * NSEM4 + b),
                              sems.at[b]).wait()
    plsc.subcore_barrier()
    pltpu.sync_copy(acc.at[pl.ds(z0, ZPT)], out.at[pl.ds(c * NPAD + z0, ZPT)])


_edge_pass = pl.kernel(
    _edge_body,
    out_type=jax.ShapeDtypeStruct((NC * NPAD, F), jnp.float32),
    mesh=_MESH,
    compiler_params=pltpu.CompilerParams(use_tc_tiling_on_sc=False),
    scratch_types=[
        pltpu.VMEM((NBLK, CHB), jnp.int32),
        pltpu.VMEM((NBLK, CHB), jnp.int32),
        pltpu.VMEM((NRING, CHB, F), jnp.float32),
        pltpu.VMEM_SHARED((NPAD, F), jnp.float32),
        pltpu.VMEM_SHARED((NPAD, F), jnp.float32),
        pltpu.SemaphoreType.DMA((NRING,)),
        pltpu.SemaphoreType.DMA((NRING,)),
    ],
)

_deg_pass = pl.kernel(
    _deg_body,
    out_type=jax.ShapeDtypeStruct((NC * NPAD,), jnp.float32),
    mesh=_MESH,
    compiler_params=pltpu.CompilerParams(use_tc_tiling_on_sc=False),
    scratch_types=[
        pltpu.VMEM((NBLK, CHB), jnp.int32),
        pltpu.VMEM((CHB,), jnp.float32),
        pltpu.VMEM_SHARED((NPAD,), jnp.float32),
        pltpu.SemaphoreType.DMA((NSEM4,)),
    ],
)


def _prep1_body(x_ref, w1_ref, degp_ref, g1_ref, r_ref):
    cnt = degp_ref[0:N_NODES, 0:1] + degp_ref[NPAD:NPAD + N_NODES, 0:1]
    r = lax.rsqrt(cnt + 1.0)
    h = jnp.dot(x_ref[...], w1_ref[...], preferred_element_type=jnp.float32)
    g1_ref[0:N_NODES] = h * r
    r_ref[...] = r


def _prep2_body(accp_ref, g1_ref, r_ref, w2p_ref, b1_ref, g2_ref):
    r = r_ref[...]
    acc = (accp_ref[0:N_NODES] + accp_ref[NPAD:NPAD + N_NODES]
           + g1_ref[0:N_NODES])
    h = jnp.maximum(acc * r + b1_ref[...], 0.0)
    g2_ref[0:N_NODES] = (
        jnp.dot(h, w2p_ref[...], preferred_element_type=jnp.float32) * r)


def _final_body(accp_ref, g2_ref, r_ref, b2p_ref, batch_ref, wfc_ref, bfc_ref,
                out_ref):
    r = r_ref[...]
    acc = (accp_ref[0:N_NODES] + accp_ref[NPAD:NPAD + N_NODES]
           + g2_ref[0:N_NODES])
    h2 = jnp.maximum(acc * r + b2p_ref[...], 0.0)[:, 0:NUM_GRAPHS]
    gids = lax.broadcasted_iota(jnp.int32, (N_NODES, NUM_GRAPHS), 1)
    oh = (batch_ref[...] == gids).astype(jnp.float32)
    pooled = lax.dot_general(oh, h2, (((0,), (0,)), ((), ())),
                             preferred_element_type=jnp.float32)
    out = pooled @ wfc_ref[...] + bfc_ref[...]
    out_ref[...] = jax.nn.sigmoid(out)


_prep1 = pl.pallas_call(
    _prep1_body,
    out_shape=[
        jax.ShapeDtypeStruct((NPAD, F), jnp.float32),
        jax.ShapeDtypeStruct((N_NODES, 1), jnp.float32),
    ],
)

_prep2 = pl.pallas_call(
    _prep2_body,
    out_shape=jax.ShapeDtypeStruct((NPAD, F), jnp.float32),
)

_final = pl.pallas_call(
    _final_body,
    out_shape=jax.ShapeDtypeStruct((NUM_GRAPHS, 1), jnp.float32),
)


def kernel(x, edge_index, batch, W1, b1, W2, b2, Wfc, bfc):
    ei = edge_index.astype(jnp.int32)
    pad = E_PAD - N_EDGES
    src3d = jnp.concatenate(
        [ei[0], jnp.zeros((pad,), jnp.int32)]).reshape(NW, NBLK, CHB)
    dst3d = jnp.concatenate(
        [ei[1], jnp.full((pad,), N_NODES, jnp.int32)]).reshape(NW, NBLK, CHB)
    zeros = jnp.zeros((NPAD, F), jnp.float32)
    zeros1 = jnp.zeros((NPAD,), jnp.float32)
    ones1 = jnp.ones((CHB,), jnp.float32)
    w2p = jnp.pad(W2, ((0, 0), (0, F - NUM_GRAPHS)))
    b1r = b1.reshape(1, F)
    b2p = jnp.pad(b2, (0, F - NUM_GRAPHS)).reshape(1, F)
    batch2d = batch.astype(jnp.int32).reshape(N_NODES, 1)

    degp = _deg_pass(ones1, dst3d, zeros1).reshape(NC * NPAD, 1)
    g1, r = _prep1(x, W1, degp)
    acc1p = _edge_pass(g1, src3d, dst3d, zeros)
    g2 = _prep2(acc1p, g1, r, w2p, b1r)
    acc2p = _edge_pass(g2, src3d, dst3d, zeros)
    return _final(acc2p, g2, r, b2p, batch2d, Wfc, bfc.reshape(1, 1))


# speedup vs baseline: 1.0214x; 1.0214x over previous
import jax
import jax.numpy as jnp
from jax import lax
from jax.experimental import pallas as pl
from jax.experimental.pallas import tpu as pltpu
from jax.experimental.pallas import tpu_sc as plsc

N_NODES = 10000
NUM_GRAPHS = 4
N_EDGES = 320000
F = 16

NC, NS = 2, 16
NW = NC * NS
BLK = 128
K = -(-N_EDGES // (NW * BLK * 8)) * 8
ROWS = K * NW
E_PAD = ROWS * BLK
NPAD = 10240
ZPT = NPAD // NS

_MESH = plsc.VectorSubcoreMesh(
    core_axis_name="c", subcore_axis_name="s", num_cores=NC, num_subcores=NS
)


CHB = 128
NBLK = K * BLK // CHB
NBUF = 4
NRING = 2 * NBUF
KS = NBLK // NRING


def _edge_body(table, src3d, dst3d, zeros, out, src_v, dst_v, bufs, acc,
               tab_sh, gsem, ssem):
    c = lax.axis_index("c")
    s = lax.axis_index("s")
    wid = c * NS + s
    z0 = s * ZPT
    pltpu.sync_copy(zeros.at[pl.ds(z0, ZPT)], acc.at[pl.ds(z0, ZPT)])
    pltpu.sync_copy(table.at[pl.ds(z0, ZPT)], tab_sh.at[pl.ds(z0, ZPT)])
    pltpu.sync_copy(src3d.at[wid], src_v)
    pltpu.sync_copy(dst3d.at[wid], dst_v)
    plsc.subcore_barrier()

    def gather(j, b):
        pltpu.async_copy(tab_sh.at[src_v.at[j]], bufs.at[b], gsem.at[b])

    def gather_wait(j, b):
        pltpu.make_async_copy(tab_sh.at[src_v.at[j]], bufs.at[b],
                              gsem.at[b]).wait()

    def scatter(j, b):
        pltpu.async_copy(bufs.at[b], acc.at[dst_v.at[j]], ssem.at[b],
                         add=True)

    def scatter_wait(j, b):
        pltpu.make_async_copy(bufs.at[b], acc.at[dst_v.at[j]],
                              ssem.at[b]).wait()

    for b in range(NBUF):
        gather(b, b)

    def step(sb, carry):
        for b in range(NRING):
            j = sb * NRING + b
            gather_wait(j, b)
            scatter(j, b)
            b4 = (b + NBUF) % NRING

            @pl.when(j >= NBUF)
            def _drain(j=j, b4=b4):
                scatter_wait(j - NBUF, b4)

            @pl.when(j + NBUF < NBLK)
            def _prefetch(j=j, b4=b4):
                gather(j + NBUF, b4)
        return carry

    lax.fori_loop(0, KS, step, 0)
    for b in range(NBUF):
        j = (KS - 1) * NRING + NBUF + b
        scatter_wait(j, (j % NRING))
    plsc.subcore_barrier()
    pltpu.sync_copy(acc.at[pl.ds(z0, ZPT)], out.at[pl.ds(c * NPAD + z0, ZPT)])


NSEM4 = 4
KS4 = NBLK // NSEM4


def _deg_body(ones, dst3d, zeros, out, dst_v, rows_v, acc, sems):
    c = lax.axis_index("c")
    s = lax.axis_index("s")
    wid = c * NS + s
    z0 = s * ZPT
    pltpu.sync_copy(zeros.at[pl.ds(z0, ZPT)], acc.at[pl.ds(z0, ZPT)])
    pltpu.sync_copy(dst3d.at[wid], dst_v)
    pltpu.sync_copy(ones, rows_v)
    plsc.subcore_barrier()

    def didx(j):
        return acc.at[dst_v.at[j]]

    def step(sb, carry):
        for b in range(NSEM4):
            j = sb * NSEM4 + b

            @pl.when(sb > 0)
            def _drain(j=j, b=b):
                pltpu.make_async_copy(rows_v, didx(j - NSEM4),
                                      sems.at[b]).wait()

            pltpu.async_copy(rows_v, didx(j), sems.at[b], add=True)
        return carry

    lax.fori_loop(0, KS4, step, 0)
    for b in range(NSEM4):
        pltpu.make_async_copy(rows_v, didx((KS4 - 1) * NSEM4 + b),
                              sems.at[b]).wait()
    plsc.subcore_barrier()
    pltpu.sync_copy(acc.at[pl.ds(z0, ZPT)], out.at[pl.ds(c * NPAD + z0, ZPT)])


_edge_pass = pl.kernel(
    _edge_body,
    out_type=jax.ShapeDtypeStruct((NC * NPAD, F), jnp.float32),
    mesh=_MESH,
    compiler_params=pltpu.CompilerParams(use_tc_tiling_on_sc=False),
    scratch_types=[
        pltpu.VMEM((NBLK, CHB), jnp.int32),
        pltpu.VMEM((NBLK, CHB), jnp.int32),
        pltpu.VMEM((NRING, CHB, F), jnp.float32),
        pltpu.VMEM_SHARED((NPAD, F), jnp.float32),
        pltpu.VMEM_SHARED((NPAD, F), jnp.float32),
        pltpu.SemaphoreType.DMA((NRING,)),
        pltpu.SemaphoreType.DMA((NRING,)),
    ],
)

_deg_pass = pl.kernel(
    _deg_body,
    out_type=jax.ShapeDtypeStruct((NC * NPAD,), jnp.float32),
    mesh=_MESH,
    compiler_params=pltpu.CompilerParams(use_tc_tiling_on_sc=False),
    scratch_types=[
        pltpu.VMEM((NBLK, CHB), jnp.int32),
        pltpu.VMEM((CHB,), jnp.float32),
        pltpu.VMEM_SHARED((NPAD,), jnp.float32),
        pltpu.SemaphoreType.DMA((NSEM4,)),
    ],
)


def _prep1_body(x_ref, w1_ref, degp_ref, g1_ref, r_ref):
    cnt = degp_ref[0:N_NODES, 0:1] + degp_ref[NPAD:NPAD + N_NODES, 0:1]
    r = lax.rsqrt(cnt + 1.0)
    h = jnp.dot(x_ref[...], w1_ref[...], preferred_element_type=jnp.float32)
    g1_ref[0:N_NODES] = h * r
    r_ref[...] = r


def _prep2_body(accp_ref, g1_ref, r_ref, w2p_ref, b1_ref, g2_ref):
    r = r_ref[...]
    acc = (accp_ref[0:N_NODES] + accp_ref[NPAD:NPAD + N_NODES]
           + g1_ref[0:N_NODES])
    h = jnp.maximum(acc * r + b1_ref[...], 0.0)
    g2_ref[0:N_NODES] = (
        jnp.dot(h, w2p_ref[...], preferred_element_type=jnp.float32) * r)


def _final_body(accp_ref, g2_ref, r_ref, b2p_ref, batch_ref, wfc_ref, bfc_ref,
                out_ref):
    r = r_ref[...]
    acc = (accp_ref[0:N_NODES] + accp_ref[NPAD:NPAD + N_NODES]
           + g2_ref[0:N_NODES])
    h2 = jnp.maximum(acc * r + b2p_ref[...], 0.0)[:, 0:NUM_GRAPHS]
    gids = lax.broadcasted_iota(jnp.int32, (N_NODES, NUM_GRAPHS), 1)
    oh = (batch_ref[...] == gids).astype(jnp.float32)
    pooled = lax.dot_general(oh, h2, (((0,), (0,)), ((), ())),
                             preferred_element_type=jnp.float32)
    out = pooled @ wfc_ref[...] + bfc_ref[...]
    out_ref[...] = jax.nn.sigmoid(out)


_prep1 = pl.pallas_call(
    _prep1_body,
    out_shape=[
        jax.ShapeDtypeStruct((NPAD, F), jnp.float32),
        jax.ShapeDtypeStruct((N_NODES, 1), jnp.float32),
    ],
)

_prep2 = pl.pallas_call(
    _prep2_body,
    out_shape=jax.ShapeDtypeStruct((NPAD, F), jnp.float32),
)

_final = pl.pallas_call(
    _final_body,
    out_shape=jax.ShapeDtypeStruct((NUM_GRAPHS, 1), jnp.float32),
)


def kernel(x, edge_index, batch, W1, b1, W2, b2, Wfc, bfc):
    ei = edge_index.astype(jnp.int32)
    pad = E_PAD - N_EDGES
    src3d = jnp.concatenate(
        [ei[0], jnp.zeros((pad,), jnp.int32)]).reshape(NW, NBLK, CHB)
    dst3d = jnp.concatenate(
        [ei[1], jnp.full((pad,), N_NODES, jnp.int32)]).reshape(NW, NBLK, CHB)
    zeros = jnp.zeros((NPAD, F), jnp.float32)
    zeros1 = jnp.zeros((NPAD,), jnp.float32)
    ones1 = jnp.ones((CHB,), jnp.float32)
    w2p = jnp.pad(W2, ((0, 0), (0, F - NUM_GRAPHS)))
    b1r = b1.reshape(1, F)
    b2p = jnp.pad(b2, (0, F - NUM_GRAPHS)).reshape(1, F)
    batch2d = batch.astype(jnp.int32).reshape(N_NODES, 1)

    degp = _deg_pass(ones1, dst3d, zeros1).reshape(NC * NPAD, 1)
    g1, r = _prep1(x, W1, degp)
    acc1p = _edge_pass(g1, src3d, dst3d, zeros)
    g2 = _prep2(acc1p, g1, r, w2p, b1r)
    acc2p = _edge_pass(g2, src3d, dst3d, zeros)
    return _final(acc2p, g2, r, b2p, batch2d, Wfc, bfc.reshape(1, 1))
